# baseline (speedup 1.0000x reference)
import functools

import jax
import jax.numpy as jnp
from jax import lax
from jax.experimental import pallas as pl
from jax.experimental.pallas import tpu as pltpu
from jax.experimental.pallas import tpu_sc as plsc

B = 16384
ED = 64
RD = 32
NC = 2
NS = 16
NW = NC * NS
RPW = B // NW
CHUNK = 128
NCHUNK = RPW // CHUNK
GRP = 16
NGRP = CHUNK // GRP
L = 16

_F32 = jnp.float32
_I32 = jnp.int32


def _rsqrt_newton(x):
    xh = x * 0.5
    i = plsc.bitcast(x, _I32)
    i = jnp.int32(0x5F3759DF) - lax.shift_right_logical(i, 1)
    y = plsc.bitcast(i, _F32)
    for _ in range(3):
        y = y * (1.5 - xh * y * y)
    return y


def _scale_from_sumsq(ss):
    return jnp.where(ss <= 1.0, jnp.float32(1.0), _rsqrt_newton(ss))


def _col_gather(ref, row_idx, j):
    return plsc.load_gather(ref, [row_idx, jnp.full((L,), j, _I32)])


def _sumsq_cols(ref, row_idx, ncols):
    ss = jnp.zeros((L,), _F32)
    for j in range(ncols):
        c = _col_gather(ref, row_idx, j)
        ss = ss + c * c
    return ss


JB = 8


def _sc_body(ph_h, pr_h, pt_h, nh_h, nr_h, nt_h, ent_h, rel_h, proj_h,
             out_h,
             phv, prv, ptv, nhv, nrv, ntv, pridx,
             hv, tv, xhv, xtv, rv, xrv,
             mbuf, dtp, dtn, lossbuf, sem):
    wid = lax.axis_index("s") * NC + lax.axis_index("c")
    wbase = wid * RPW
    iota = lax.iota(_I32, L)

    def chunk_body(c, loss_vec):
        cb = wbase + c * CHUNK
        pltpu.sync_copy(ph_h.at[pl.ds(cb, CHUNK)], phv)
        pltpu.sync_copy(pr_h.at[pl.ds(cb, CHUNK)], prv)
        pltpu.sync_copy(pt_h.at[pl.ds(cb, CHUNK)], ptv)
        pltpu.sync_copy(nh_h.at[pl.ds(cb, CHUNK)], nhv)
        pltpu.sync_copy(nr_h.at[pl.ds(cb, CHUNK)], nrv)
        pltpu.sync_copy(nt_h.at[pl.ds(cb, CHUNK)], ntv)
        cp1 = pltpu.async_copy(ent_h.at[phv], hv, sem)
        cp2 = pltpu.async_copy(ent_h.at[ptv], tv, sem)
        cp3 = pltpu.async_copy(ent_h.at[nhv], xhv, sem)
        cp4 = pltpu.async_copy(ent_h.at[ntv], xtv, sem)
        cp5 = pltpu.async_copy(rel_h.at[prv], rv, sem)
        cp6 = pltpu.async_copy(rel_h.at[nrv], xrv, sem)
        cp1.wait(); cp2.wait(); cp3.wait(); cp4.wait(); cp5.wait(); cp6.wait()

        def group_body(g, loss_vec_g):
            r0 = g * GRP
            pridx[...] = prv[pl.ds(r0, GRP)]
            pltpu.async_copy(proj_h.at[pridx], mbuf, sem).wait()

            row_idx = r0 + iota
            s_h = _scale_from_sumsq(_sumsq_cols(hv, row_idx, ED))
            s_t = _scale_from_sumsq(_sumsq_cols(tv, row_idx, ED))
            s_xh = _scale_from_sumsq(_sumsq_cols(xhv, row_idx, ED))
            s_xt = _scale_from_sumsq(_sumsq_cols(xtv, row_idx, ED))
            s_r = _scale_from_sumsq(_sumsq_cols(rv, row_idx, RD))
            s_xr = _scale_from_sumsq(_sumsq_cols(xrv, row_idx, RD))
            for k in range(ED):
                ch = _col_gather(hv, row_idx, k)
                ct = _col_gather(tv, row_idx, k)
                dtp[k] = s_h * ch - s_t * ct
                cxh = _col_gather(xhv, row_idx, k)
                cxt = _col_gather(xtv, row_idx, k)
                dtn[k] = s_xh * cxh - s_xt * cxt

            q = jnp.zeros((L,), _F32)
            for jb in range(RD // JB):
                def k_body(k, accs):
                    dp = dtp[k]
                    dn = dtn[k]
                    new = []
                    for jj in range(JB):
                        col = k * RD + jb * JB + jj
                        cv = jnp.zeros((L,), _I32) + col
                        mg = plsc.load_gather(mbuf, [iota, cv])
                        new.append((accs[2 * jj] + dp * mg,
                                    accs[2 * jj + 1] + dn * mg))
                    return tuple(x for pair in new for x in pair)

                accs = lax.fori_loop(
                    0, ED, k_body,
                    tuple(jnp.zeros((L,), _F32) for _ in range(2 * JB)))
                for jj in range(JB):
                    j = jb * JB + jj
                    rc = _col_gather(rv, row_idx, j)
                    xc = _col_gather(xrv, row_idx, j)
                    vp = accs[2 * jj] + s_r * rc
                    vn = accs[2 * jj + 1] + s_xr * xc
                    q = q + vp * vp - vn * vn

            term = jnp.maximum(q + 1.0, 0.0)
            return loss_vec_g + term

        return lax.fori_loop(0, NGRP, group_body, loss_vec)

    loss_vec = lax.fori_loop(0, NCHUNK, chunk_body, jnp.zeros((L,), _F32))
    lossbuf[...] = loss_vec
    pltpu.sync_copy(lossbuf, out_h.at[wid])


_sc_kernel = functools.partial(
    pl.kernel,
    mesh=plsc.VectorSubcoreMesh(core_axis_name="c", subcore_axis_name="s"),
    out_type=jax.ShapeDtypeStruct((NW, L), _F32),
    compiler_params=pltpu.CompilerParams(
        needs_layout_passes=False, use_tc_tiling_on_sc=False),
    scratch_types=[
        pltpu.VMEM((CHUNK,), _I32),
        pltpu.VMEM((CHUNK,), _I32),
        pltpu.VMEM((CHUNK,), _I32),
        pltpu.VMEM((CHUNK,), _I32),
        pltpu.VMEM((CHUNK,), _I32),
        pltpu.VMEM((CHUNK,), _I32),
        pltpu.VMEM((GRP,), _I32),
        pltpu.VMEM((CHUNK, ED), _F32),
        pltpu.VMEM((CHUNK, ED), _F32),
        pltpu.VMEM((CHUNK, ED), _F32),
        pltpu.VMEM((CHUNK, ED), _F32),
        pltpu.VMEM((CHUNK, RD), _F32),
        pltpu.VMEM((CHUNK, RD), _F32),
        pltpu.VMEM((GRP, ED * RD), _F32),
        pltpu.VMEM((ED, L), _F32),
        pltpu.VMEM((ED, L), _F32),
        pltpu.VMEM((L,), _F32),
        pltpu.SemaphoreType.DMA,
    ],
)(_sc_body)


def kernel(batch_positives, batch_negatives, entity_emb, relation_emb,
           proj_emb):
    ph = batch_positives[:, 0]
    pr = batch_positives[:, 1]
    pt = batch_positives[:, 2]
    nh = batch_negatives[:, 0]
    nr = batch_negatives[:, 1]
    nt = batch_negatives[:, 2]
    partials = _sc_kernel(ph, pr, pt, nh, nr, nt, entity_emb, relation_emb,
                          proj_emb)
    return jnp.sum(partials) / jnp.float32(B)
